# baseline (speedup 1.0000x reference)
import jax
import jax.numpy as jnp
from jax.experimental import pallas as pl

TOPK = 8
N_TOKENS = 32768
D_MODEL = 4096
N_EXPERTS = 64
BT = 1024


def _router_body(x_ref, wt_ref, idx_ref, probs_ref, gate_ref, logits_ref):
    x = x_ref[...]
    wt = wt_ref[...]
    logits = jnp.dot(x, wt, preferred_element_type=jnp.float32)
    logits_ref[...] = logits

    lt = logits.T
    m = jnp.max(lt, axis=0, keepdims=True)
    et = jnp.exp(lt - m)
    s = jnp.sum(et, axis=0, keepdims=True)
    probs_ref[...] = (et / s).T

    rows = jax.lax.broadcasted_iota(jnp.int32, et.shape, 0)
    work = et
    vals = []
    idxs = []
    for _ in range(TOPK):
        mx = jnp.max(work, axis=0, keepdims=True)
        ix = jnp.min(jnp.where(work == mx, rows, N_EXPERTS), axis=0,
                     keepdims=True)
        vals.append(mx)
        idxs.append(ix)
        work = jnp.where(rows == ix, -1.0, work)

    vals_t = jnp.concatenate(vals, axis=0)
    gate_t = vals_t / jnp.sum(vals_t, axis=0, keepdims=True)

    gate_ref[...] = gate_t.T
    idx_ref[...] = jnp.concatenate(idxs, axis=0).T


@jax.jit
def kernel(x, W):
    wt = W.T
    grid = (N_TOKENS // BT,)
    out_shapes = (
        jax.ShapeDtypeStruct((N_TOKENS, TOPK), jnp.int32),
        jax.ShapeDtypeStruct((N_TOKENS, N_EXPERTS), jnp.float32),
        jax.ShapeDtypeStruct((N_TOKENS, TOPK), jnp.float32),
        jax.ShapeDtypeStruct((N_TOKENS, N_EXPERTS), jnp.float32),
    )
    topk_idx, probs, gate, logits = pl.pallas_call(
        _router_body,
        grid=grid,
        in_specs=[
            pl.BlockSpec((BT, D_MODEL), lambda i: (i, 0)),
            pl.BlockSpec((D_MODEL, N_EXPERTS), lambda i: (0, 0)),
        ],
        out_specs=(
            pl.BlockSpec((BT, TOPK), lambda i: (i, 0)),
            pl.BlockSpec((BT, N_EXPERTS), lambda i: (i, 0)),
            pl.BlockSpec((BT, TOPK), lambda i: (i, 0)),
            pl.BlockSpec((BT, N_EXPERTS), lambda i: (i, 0)),
        ),
        out_shape=out_shapes,
    )(x, wt)
    return (topk_idx, probs, gate, logits)


# speedup vs baseline: 1.0349x; 1.0349x over previous
import jax
import jax.numpy as jnp
from jax.experimental import pallas as pl

TOPK = 8
N_TOKENS = 32768
D_MODEL = 4096
N_EXPERTS = 64
BT = 1024


def _router_body(x_ref, w_ref, idx_ref, probs_ref, gate_ref, logits_ref):
    x = x_ref[...]
    w = w_ref[...]
    logits = jax.lax.dot_general(
        x, w, (((1,), (1,)), ((), ())),
        preferred_element_type=jnp.float32)
    logits_ref[...] = logits

    lt = logits.T
    m = jnp.max(lt, axis=0, keepdims=True)
    et = jnp.exp(lt - m)
    s = jnp.sum(et, axis=0, keepdims=True)
    probs_ref[...] = (et / s).T

    rows = jax.lax.broadcasted_iota(jnp.int32, et.shape, 0)
    work = et
    vals = []
    idxs = []
    for _ in range(TOPK):
        mx = jnp.max(work, axis=0, keepdims=True)
        ix = jnp.min(jnp.where(work == mx, rows, N_EXPERTS), axis=0,
                     keepdims=True)
        vals.append(mx)
        idxs.append(ix)
        work = jnp.where(rows == ix, -1.0, work)

    vals_t = jnp.concatenate(vals, axis=0)
    gate_t = vals_t / jnp.sum(vals_t, axis=0, keepdims=True)

    gate_ref[...] = gate_t.T
    idx_ref[...] = jnp.concatenate(idxs, axis=0).T


@jax.jit
def kernel(x, W):
    grid = (N_TOKENS // BT,)
    out_shapes = (
        jax.ShapeDtypeStruct((N_TOKENS, TOPK), jnp.int32),
        jax.ShapeDtypeStruct((N_TOKENS, N_EXPERTS), jnp.float32),
        jax.ShapeDtypeStruct((N_TOKENS, TOPK), jnp.float32),
        jax.ShapeDtypeStruct((N_TOKENS, N_EXPERTS), jnp.float32),
    )
    topk_idx, probs, gate, logits = pl.pallas_call(
        _router_body,
        grid=grid,
        in_specs=[
            pl.BlockSpec((BT, D_MODEL), lambda i: (i, 0)),
            pl.BlockSpec((N_EXPERTS, D_MODEL), lambda i: (0, 0)),
        ],
        out_specs=(
            pl.BlockSpec((BT, TOPK), lambda i: (i, 0)),
            pl.BlockSpec((BT, N_EXPERTS), lambda i: (i, 0)),
            pl.BlockSpec((BT, TOPK), lambda i: (i, 0)),
            pl.BlockSpec((BT, N_EXPERTS), lambda i: (i, 0)),
        ),
        out_shape=out_shapes,
    )(x, W)
    return (topk_idx, probs, gate, logits)
